# baseline (speedup 1.0000x reference)
import functools

import jax
import jax.numpy as jnp
from jax import lax
from jax.experimental import pallas as pl
from jax.experimental.pallas import tpu as pltpu
from jax.experimental.pallas import tpu_sc as plsc

N = 10000
E = 320000
H = 128
G = 64
EPS = 1e-5

NCORE = 2
NSUB = 16
NW = NCORE * NSUB
NPAD = 10240
ROWS_PER_W = NPAD // NW
GCHUNKS = ((0, 128), (128, 128), (256, 64))
EPW = 80 * 128
EPAD = NW * EPW
ECHUNK = 128
NSTEPS = EPW // ECHUNK
NBUF = 2
IBLK = 16
NIBLK = NSTEPS // IBLK
DUMMY = 10200
RPT = NPAD // NSUB
DPT = RPT

_mesh = plsc.VectorSubcoreMesh(core_axis_name="c", subcore_axis_name="s")


def _zero_rows(ref, nrows):
    def body(r, _):
        for k in range(H // 16):
            ref[r, pl.ds(k * 16, 16)] = jnp.zeros((16,), jnp.float32)
        return 0
    lax.fori_loop(0, nrows, body, 0)


def _zero_vec(ref, n):
    def body(r, _):
        ref[pl.ds(r * 16, 16)] = jnp.zeros((16,), jnp.float32)
        return 0
    lax.fori_loop(0, n // 16, body, 0)


def _prep_body(nf2, emb, src3, dst3, h0, odp, idp,
               nf_v, src_v, dst_v, rows_v, ones_v, dbuf_v, od_s, id_s, sem, dsem):
    c = lax.axis_index("c")
    s = lax.axis_index("s")
    wid = c * NSUB + s

    pltpu.sync_copy(nf2.at[wid], nf_v)
    pltpu.sync_copy(src3.at[wid], src_v)
    pltpu.sync_copy(dst3.at[wid], dst_v)

    _zero_vec(dbuf_v, DPT)
    pltpu.sync_copy(dbuf_v, od_s.at[pl.ds(s * DPT, DPT)])
    pltpu.sync_copy(dbuf_v, id_s.at[pl.ds(s * DPT, DPT)])
    for k in range(H // 16):
        ones_v[pl.ds(k * 16, 16)] = jnp.full((16,), 1.0, jnp.float32)
    plsc.subcore_barrier()

    def dfire(j, _):
        pltpu.async_copy(ones_v, od_s.at[src_v.at[j]], dsem, add=True)
        pltpu.async_copy(ones_v, id_s.at[dst_v.at[j]], dsem, add=True)
        return 0
    lax.fori_loop(0, NSTEPS, dfire, 0)

    for off, sz in GCHUNKS:
        pltpu.async_copy(emb.at[nf_v.at[pl.ds(off, sz)]],
                         rows_v.at[pl.ds(0, sz)], sem).wait()
        pltpu.sync_copy(rows_v.at[pl.ds(0, sz)],
                        h0.at[pl.ds(wid * ROWS_PER_W + off, sz)])

    def ddrain(j, _):
        pltpu.make_async_copy(ones_v, od_s.at[src_v.at[j]], dsem).wait()
        pltpu.make_async_copy(ones_v, id_s.at[dst_v.at[j]], dsem).wait()
        return 0
    lax.fori_loop(0, NSTEPS, ddrain, 0)

    plsc.subcore_barrier()

    pltpu.sync_copy(od_s.at[pl.ds(s * DPT, DPT)], dbuf_v)
    pltpu.sync_copy(dbuf_v, odp.at[c, pl.ds(s * DPT, DPT)])
    pltpu.sync_copy(id_s.at[pl.ds(s * DPT, DPT)], dbuf_v)
    pltpu.sync_copy(dbuf_v, idp.at[c, pl.ds(s * DPT, DPT)])


_prep_call = pl.kernel(
    _prep_body,
    out_type=(
        jax.ShapeDtypeStruct((NPAD, H), jnp.float32),
        jax.ShapeDtypeStruct((NCORE, NPAD), jnp.float32),
        jax.ShapeDtypeStruct((NCORE, NPAD), jnp.float32),
    ),
    mesh=_mesh,
    scratch_types=[
        pltpu.VMEM((ROWS_PER_W,), jnp.int32),
        pltpu.VMEM((NSTEPS, ECHUNK), jnp.int32),
        pltpu.VMEM((NSTEPS, ECHUNK), jnp.int32),
        pltpu.VMEM((ECHUNK, H), jnp.float32),
        pltpu.VMEM((ECHUNK,), jnp.float32),
        pltpu.VMEM((DPT,), jnp.float32),
        pltpu.VMEM_SHARED((NPAD,), jnp.float32),
        pltpu.VMEM_SHARED((NPAD,), jnp.float32),
        pltpu.SemaphoreType.DMA,
        pltpu.SemaphoreType.DMA,
    ],
)


def _agg_body(hs, src3, dst3, part, src_i, dst_i, rows_v, agg_s, sem):
    c = lax.axis_index("c")
    s = lax.axis_index("s")
    wid = c * NSUB + s

    _zero_rows(rows_v.at[0], ECHUNK)
    for i in range(RPT // ECHUNK):
        pltpu.sync_copy(rows_v.at[0], agg_s.at[pl.ds(s * RPT + i * ECHUNK, ECHUNK)])
    plsc.subcore_barrier()

    for blk in range(NIBLK):
        pltpu.sync_copy(src3.at[wid, pl.ds(blk * IBLK, IBLK)], src_i)
        pltpu.sync_copy(dst3.at[wid, pl.ds(blk * IBLK, IBLK)], dst_i)
        for b in range(NBUF):
            pltpu.async_copy(hs.at[src_i.at[b]], rows_v.at[b], sem)

        def outer(g, _):
            for b in range(NBUF):
                j = g * NBUF + b
                pltpu.make_async_copy(hs.at[src_i.at[j]], rows_v.at[b], sem).wait()
                pltpu.sync_copy(rows_v.at[b], agg_s.at[dst_i.at[j]], add=True)

                @pl.when(j + NBUF < IBLK)
                def _():
                    pltpu.async_copy(hs.at[src_i.at[j + NBUF]], rows_v.at[b], sem)
            return 0
        lax.fori_loop(0, IBLK // NBUF, outer, 0)

    plsc.subcore_barrier()

    for i in range(RPT // ECHUNK):
        pltpu.sync_copy(agg_s.at[pl.ds(s * RPT + i * ECHUNK, ECHUNK)], rows_v.at[0])
        pltpu.sync_copy(rows_v.at[0], part.at[c, pl.ds(s * RPT + i * ECHUNK, ECHUNK)])


_agg_call = pl.kernel(
    _agg_body,
    out_type=jax.ShapeDtypeStruct((NCORE, NPAD, H), jnp.float32),
    mesh=_mesh,
    scratch_types=[
        pltpu.VMEM((IBLK, ECHUNK), jnp.int32),
        pltpu.VMEM((IBLK, ECHUNK), jnp.int32),
        pltpu.VMEM((NBUF, ECHUNK, H), jnp.float32),
        pltpu.VMEM_SHARED((NPAD, H), jnp.float32),
        pltpu.SemaphoreType.DMA,
    ],
)


_BLK = 1024
_NBLK = NPAD // _BLK


def _scale_body(odp_ref, idp_ref, h0_ref, hs_ref, osc_ref, isc_ref):
    od = odp_ref[...]
    idg = idp_ref[...]
    osc = lax.rsqrt(jnp.maximum(od[:, 0:1] + od[:, 1:2], 1.0))
    isc = lax.rsqrt(jnp.maximum(idg[:, 0:1] + idg[:, 1:2], 1.0))
    hs_ref[...] = h0_ref[...] * osc
    osc_ref[...] = osc
    isc_ref[...] = isc


_scale_call = pl.pallas_call(
    _scale_body,
    grid=(_NBLK,),
    in_specs=[
        pl.BlockSpec((_BLK, 2), lambda i: (i, 0)),
        pl.BlockSpec((_BLK, 2), lambda i: (i, 0)),
        pl.BlockSpec((_BLK, H), lambda i: (i, 0)),
    ],
    out_specs=[
        pl.BlockSpec((_BLK, H), lambda i: (i, 0)),
        pl.BlockSpec((_BLK, 1), lambda i: (i, 0)),
        pl.BlockSpec((_BLK, 1), lambda i: (i, 0)),
    ],
    out_shape=[
        jax.ShapeDtypeStruct((NPAD, H), jnp.float32),
        jax.ShapeDtypeStruct((NPAD, 1), jnp.float32),
        jax.ShapeDtypeStruct((NPAD, 1), jnp.float32),
    ],
)


def _layer_body(part_ref, isc_ref, osc_ref, w_ref, b_ref, out_ref):
    p = part_ref[0] + part_ref[1]
    agg = p * isc_ref[...]
    h = lax.dot_general(agg, w_ref[...], (((1,), (0,)), ((), ())),
                        preferred_element_type=jnp.float32)
    h = jnp.maximum(h + b_ref[...], 0.0)
    out_ref[...] = h * osc_ref[...]


_layer1_call = pl.pallas_call(
    _layer_body,
    grid=(_NBLK,),
    in_specs=[
        pl.BlockSpec((NCORE, _BLK, H), lambda i: (0, i, 0)),
        pl.BlockSpec((_BLK, 1), lambda i: (i, 0)),
        pl.BlockSpec((_BLK, 1), lambda i: (i, 0)),
        pl.BlockSpec((H, H), lambda i: (0, 0)),
        pl.BlockSpec((1, H), lambda i: (0, 0)),
    ],
    out_specs=pl.BlockSpec((_BLK, H), lambda i: (i, 0)),
    out_shape=jax.ShapeDtypeStruct((NPAD, H), jnp.float32),
)


def _leaky(x, slope):
    return jnp.where(x >= 0, x, slope * x)


def _final_body(part_ref, isc_ref, gid_ref, w2_ref, b2_ref,
                we1_ref, be1_ref, g0_ref, bb0_ref,
                we2_ref, be2_ref, g1_ref, bb1_ref,
                we3_ref, be3_ref, we4_ref, be4_ref, we5_ref, be5_ref,
                out_ref, hg_acc, cnt_acc):
    i = pl.program_id(0)

    p = part_ref[0] + part_ref[1]
    agg = p * isc_ref[...]
    h2 = lax.dot_general(agg, w2_ref[...], (((1,), (0,)), ((), ())),
                         preferred_element_type=jnp.float32)
    h2 = jnp.maximum(h2 + b2_ref[...], 0.0)

    ids = gid_ref[...]
    seg = lax.broadcasted_iota(jnp.int32, (_BLK, G), 1)
    oh = (ids == seg).astype(jnp.float32)

    @pl.when(i == 0)
    def _():
        hg_acc[...] = jnp.zeros((G, H), jnp.float32)
        cnt_acc[...] = jnp.zeros((G, 1), jnp.float32)

    hg_acc[...] += lax.dot_general(oh, h2, (((0,), (0,)), ((), ())),
                                   preferred_element_type=jnp.float32)
    cnt_acc[...] += lax.dot_general(oh, jnp.ones((_BLK, 1), jnp.float32),
                                    (((0,), (0,)), ((), ())),
                                    preferred_element_type=jnp.float32)

    @pl.when(i == _NBLK - 1)
    def _():
        hg = hg_acc[...] / jnp.maximum(cnt_acc[...], 1.0)
        bninv = 1.0 / jnp.sqrt(1.0 + EPS)

        def lin_t(x, w_ref, b_ref):
            return lax.dot_general(x, w_ref[...], (((1,), (1,)), ((), ())),
                                   preferred_element_type=jnp.float32) + b_ref[...]

        h1 = _leaky(g0_ref[...] * lin_t(hg, we1_ref, be1_ref) * bninv + bb0_ref[...], 0.05)
        hb = _leaky(g1_ref[...] * lin_t(h1, we2_ref, be2_ref) * bninv + bb1_ref[...], 0.05)
        h3 = _leaky(lin_t(hb, we3_ref, be3_ref), 0.1)
        h4 = _leaky(lin_t(h3, we4_ref, be4_ref), 0.1)
        y = lin_t(h4, we5_ref, be5_ref)
        m = jnp.max(y, axis=1, keepdims=True)
        z = y - m
        out_ref[...] = z - jnp.log(jnp.sum(jnp.exp(z), axis=1, keepdims=True))


_final_call = pl.pallas_call(
    _final_body,
    grid=(_NBLK,),
    in_specs=[
        pl.BlockSpec((NCORE, _BLK, H), lambda i: (0, i, 0)),
        pl.BlockSpec((_BLK, 1), lambda i: (i, 0)),
        pl.BlockSpec((_BLK, 1), lambda i: (i, 0)),
        pl.BlockSpec((H, H), lambda i: (0, 0)),
        pl.BlockSpec((1, H), lambda i: (0, 0)),
        pl.BlockSpec((H, H), lambda i: (0, 0)),
        pl.BlockSpec((1, H), lambda i: (0, 0)),
        pl.BlockSpec((1, H), lambda i: (0, 0)),
        pl.BlockSpec((1, H), lambda i: (0, 0)),
        pl.BlockSpec((H, H), lambda i: (0, 0)),
        pl.BlockSpec((1, H), lambda i: (0, 0)),
        pl.BlockSpec((1, H), lambda i: (0, 0)),
        pl.BlockSpec((1, H), lambda i: (0, 0)),
        pl.BlockSpec((G, H), lambda i: (0, 0)),
        pl.BlockSpec((1, G), lambda i: (0, 0)),
        pl.BlockSpec((32, G), lambda i: (0, 0)),
        pl.BlockSpec((1, 32), lambda i: (0, 0)),
        pl.BlockSpec((10, 32), lambda i: (0, 0)),
        pl.BlockSpec((1, 10), lambda i: (0, 0)),
    ],
    out_specs=pl.BlockSpec((G, 10), lambda i: (0, 0)),
    out_shape=jax.ShapeDtypeStruct((G, 10), jnp.float32),
    scratch_shapes=[
        pltpu.VMEM((G, H), jnp.float32),
        pltpu.VMEM((G, 1), jnp.float32),
    ],
)


def kernel(node_feat, edge_index, graph_ids, emb, W1, b1, W2, b2,
           We1, be1, g0, bb0, We2, be2, g1, bb1, We3, be3, We4, be4, We5, be5):
    nf2 = jnp.concatenate(
        [node_feat, jnp.zeros((NPAD - N,), jnp.int32)]).reshape(NW, ROWS_PER_W)
    pad_e = jnp.full((EPAD - E,), DUMMY, jnp.int32)
    src3 = jnp.concatenate([edge_index[0], pad_e]).reshape(NW, NSTEPS, ECHUNK)
    dst3 = jnp.concatenate([edge_index[1], pad_e]).reshape(NW, NSTEPS, ECHUNK)
    gid = jnp.concatenate(
        [graph_ids, jnp.full((NPAD - N,), G, jnp.int32)]).reshape(NPAD, 1)

    h0, odp, idp = _prep_call(nf2, emb, src3, dst3)
    hs0, osc, isc = _scale_call(jnp.transpose(odp), jnp.transpose(idp), h0)

    part1 = _agg_call(hs0, src3, dst3)
    hs1 = _layer1_call(part1, isc, osc, W1, b1.reshape(1, H))

    part2 = _agg_call(hs1, src3, dst3)
    out = _final_call(part2, isc, gid, W2, b2.reshape(1, H),
                      We1, be1.reshape(1, H), g0.reshape(1, H), bb0.reshape(1, H),
                      We2, be2.reshape(1, H), g1.reshape(1, H), bb1.reshape(1, H),
                      We3, be3.reshape(1, G), We4, be4.reshape(1, 32),
                      We5, be5.reshape(1, 10))
    return out


# speedup vs baseline: 3.1008x; 3.1008x over previous
import functools

import jax
import jax.numpy as jnp
from jax import lax
from jax.experimental import pallas as pl
from jax.experimental.pallas import tpu as pltpu
from jax.experimental.pallas import tpu_sc as plsc

N = 10000
E = 320000
H = 128
G = 64
EPS = 1e-5

NCORE = 2
NSUB = 16
NW = NCORE * NSUB
NPAD = 10240
ROWS_PER_W = NPAD // NW
GCHUNKS = ((0, 128), (128, 128), (256, 64))
EPW = 80 * 128
EPAD = NW * EPW
ECHUNK = 128
NSTEPS = EPW // ECHUNK
NBUF = 2
IBLK = 16
NIBLK = NSTEPS // IBLK
DUMMY = 10200
RPT = NPAD // NSUB
DPT = RPT

_mesh = plsc.VectorSubcoreMesh(core_axis_name="c", subcore_axis_name="s")


def _zero_rows(ref, nrows):
    def body(r, _):
        for k in range(H // 16):
            ref[r, pl.ds(k * 16, 16)] = jnp.zeros((16,), jnp.float32)
        return 0
    lax.fori_loop(0, nrows, body, 0)


def _zero_vec(ref, n):
    def body(r, _):
        ref[pl.ds(r * 16, 16)] = jnp.zeros((16,), jnp.float32)
        return 0
    lax.fori_loop(0, n // 16, body, 0)


def _prep_body(nf2, emb, src3, dst3, h0, odp, idp,
               nf_v, src_v, dst_v, rows_v, ones_v, dbuf_v, od_s, id_s, sem, dsem):
    c = lax.axis_index("c")
    s = lax.axis_index("s")
    wid = c * NSUB + s

    pltpu.sync_copy(nf2.at[wid], nf_v)
    pltpu.sync_copy(src3.at[wid], src_v)
    pltpu.sync_copy(dst3.at[wid], dst_v)

    _zero_vec(dbuf_v, DPT)
    pltpu.sync_copy(dbuf_v, od_s.at[pl.ds(s * DPT, DPT)])
    pltpu.sync_copy(dbuf_v, id_s.at[pl.ds(s * DPT, DPT)])
    for k in range(H // 16):
        ones_v[pl.ds(k * 16, 16)] = jnp.full((16,), 1.0, jnp.float32)
    plsc.subcore_barrier()

    def dfire(j, _):
        pltpu.async_copy(ones_v, od_s.at[src_v.at[j]], dsem, add=True)
        pltpu.async_copy(ones_v, id_s.at[dst_v.at[j]], dsem, add=True)
        return 0
    lax.fori_loop(0, NSTEPS, dfire, 0)

    for off, sz in GCHUNKS:
        pltpu.async_copy(emb.at[nf_v.at[pl.ds(off, sz)]],
                         rows_v.at[pl.ds(0, sz)], sem).wait()
        pltpu.sync_copy(rows_v.at[pl.ds(0, sz)],
                        h0.at[pl.ds(wid * ROWS_PER_W + off, sz)])

    def ddrain(j, _):
        pltpu.make_async_copy(ones_v, od_s.at[src_v.at[j]], dsem).wait()
        pltpu.make_async_copy(ones_v, id_s.at[dst_v.at[j]], dsem).wait()
        return 0
    lax.fori_loop(0, NSTEPS, ddrain, 0)

    plsc.subcore_barrier()

    pltpu.sync_copy(od_s.at[pl.ds(s * DPT, DPT)], dbuf_v)
    pltpu.sync_copy(dbuf_v, odp.at[c, pl.ds(s * DPT, DPT)])
    pltpu.sync_copy(id_s.at[pl.ds(s * DPT, DPT)], dbuf_v)
    pltpu.sync_copy(dbuf_v, idp.at[c, pl.ds(s * DPT, DPT)])


_prep_call = pl.kernel(
    _prep_body,
    out_type=(
        jax.ShapeDtypeStruct((NPAD, H), jnp.float32),
        jax.ShapeDtypeStruct((NCORE, NPAD), jnp.float32),
        jax.ShapeDtypeStruct((NCORE, NPAD), jnp.float32),
    ),
    mesh=_mesh,
    scratch_types=[
        pltpu.VMEM((ROWS_PER_W,), jnp.int32),
        pltpu.VMEM((NSTEPS, ECHUNK), jnp.int32),
        pltpu.VMEM((NSTEPS, ECHUNK), jnp.int32),
        pltpu.VMEM((ECHUNK, H), jnp.float32),
        pltpu.VMEM((ECHUNK,), jnp.float32),
        pltpu.VMEM((DPT,), jnp.float32),
        pltpu.VMEM_SHARED((NPAD,), jnp.float32),
        pltpu.VMEM_SHARED((NPAD,), jnp.float32),
        pltpu.SemaphoreType.DMA,
        pltpu.SemaphoreType.DMA,
    ],
)


def _agg_body(hs, src3, dst3, part, src_i, dst_i, rows_v, agg_s, sem):
    c = lax.axis_index("c")
    s = lax.axis_index("s")
    wid = c * NSUB + s

    _zero_rows(rows_v.at[0], ECHUNK)
    for i in range(RPT // ECHUNK):
        pltpu.sync_copy(rows_v.at[0], agg_s.at[pl.ds(s * RPT + i * ECHUNK, ECHUNK)])
    plsc.subcore_barrier()

    for blk in range(NIBLK):
        pltpu.sync_copy(src3.at[wid, pl.ds(blk * IBLK, IBLK)], src_i)
        pltpu.sync_copy(dst3.at[wid, pl.ds(blk * IBLK, IBLK)], dst_i)
        for b in range(NBUF):
            pltpu.async_copy(hs.at[src_i.at[b]], rows_v.at[b], sem)

        def outer(g, _):
            for b in range(NBUF):
                j = g * NBUF + b
                pltpu.make_async_copy(hs.at[src_i.at[j]], rows_v.at[b], sem).wait()
                pltpu.sync_copy(rows_v.at[b], agg_s.at[dst_i.at[j]], add=True)

                @pl.when(j + NBUF < IBLK)
                def _():
                    pltpu.async_copy(hs.at[src_i.at[j + NBUF]], rows_v.at[b], sem)
            return 0
        lax.fori_loop(0, IBLK // NBUF, outer, 0)

    plsc.subcore_barrier()

    for i in range(RPT // ECHUNK):
        pltpu.sync_copy(agg_s.at[pl.ds(s * RPT + i * ECHUNK, ECHUNK)], rows_v.at[0])
        pltpu.sync_copy(rows_v.at[0], part.at[c, pl.ds(s * RPT + i * ECHUNK, ECHUNK)])


_agg_call = pl.kernel(
    _agg_body,
    out_type=jax.ShapeDtypeStruct((NCORE, NPAD, H), jnp.float32),
    mesh=_mesh,
    scratch_types=[
        pltpu.VMEM((IBLK, ECHUNK), jnp.int32),
        pltpu.VMEM((IBLK, ECHUNK), jnp.int32),
        pltpu.VMEM((NBUF, ECHUNK, H), jnp.float32),
        pltpu.VMEM_SHARED((NPAD, H), jnp.float32),
        pltpu.SemaphoreType.DMA,
    ],
)


_BLK = 1024
_NBLK = NPAD // _BLK


def _scale_body(odp_ref, idp_ref, h0_ref, hs_ref, osc_ref, isc_ref):
    od = odp_ref[...]
    idg = idp_ref[...]
    osc = lax.rsqrt(jnp.maximum(od[:, 0:1] + od[:, 1:2], 1.0))
    isc = lax.rsqrt(jnp.maximum(idg[:, 0:1] + idg[:, 1:2], 1.0))
    hs_ref[...] = h0_ref[...] * osc
    osc_ref[...] = osc
    isc_ref[...] = isc


_scale_call = pl.pallas_call(
    _scale_body,
    grid=(_NBLK,),
    in_specs=[
        pl.BlockSpec((_BLK, 2), lambda i: (i, 0)),
        pl.BlockSpec((_BLK, 2), lambda i: (i, 0)),
        pl.BlockSpec((_BLK, H), lambda i: (i, 0)),
    ],
    out_specs=[
        pl.BlockSpec((_BLK, H), lambda i: (i, 0)),
        pl.BlockSpec((_BLK, 1), lambda i: (i, 0)),
        pl.BlockSpec((_BLK, 1), lambda i: (i, 0)),
    ],
    out_shape=[
        jax.ShapeDtypeStruct((NPAD, H), jnp.float32),
        jax.ShapeDtypeStruct((NPAD, 1), jnp.float32),
        jax.ShapeDtypeStruct((NPAD, 1), jnp.float32),
    ],
)


def _layer_body(part_ref, isc_ref, osc_ref, w_ref, b_ref, out_ref):
    p = part_ref[0] + part_ref[1]
    agg = p * isc_ref[...]
    h = lax.dot_general(agg, w_ref[...], (((1,), (0,)), ((), ())),
                        preferred_element_type=jnp.float32)
    h = jnp.maximum(h + b_ref[...], 0.0)
    out_ref[...] = h * osc_ref[...]


_layer1_call = pl.pallas_call(
    _layer_body,
    grid=(_NBLK,),
    in_specs=[
        pl.BlockSpec((NCORE, _BLK, H), lambda i: (0, i, 0)),
        pl.BlockSpec((_BLK, 1), lambda i: (i, 0)),
        pl.BlockSpec((_BLK, 1), lambda i: (i, 0)),
        pl.BlockSpec((H, H), lambda i: (0, 0)),
        pl.BlockSpec((1, H), lambda i: (0, 0)),
    ],
    out_specs=pl.BlockSpec((_BLK, H), lambda i: (i, 0)),
    out_shape=jax.ShapeDtypeStruct((NPAD, H), jnp.float32),
)


def _leaky(x, slope):
    return jnp.where(x >= 0, x, slope * x)


def _final_body(part_ref, isc_ref, gid_ref, w2_ref, b2_ref,
                we1_ref, be1_ref, g0_ref, bb0_ref,
                we2_ref, be2_ref, g1_ref, bb1_ref,
                we3_ref, be3_ref, we4_ref, be4_ref, we5_ref, be5_ref,
                out_ref, hg_acc, cnt_acc):
    i = pl.program_id(0)

    p = part_ref[0] + part_ref[1]
    agg = p * isc_ref[...]
    h2 = lax.dot_general(agg, w2_ref[...], (((1,), (0,)), ((), ())),
                         preferred_element_type=jnp.float32)
    h2 = jnp.maximum(h2 + b2_ref[...], 0.0)

    ids = gid_ref[...]
    seg = lax.broadcasted_iota(jnp.int32, (_BLK, G), 1)
    oh = (ids == seg).astype(jnp.float32)

    @pl.when(i == 0)
    def _():
        hg_acc[...] = jnp.zeros((G, H), jnp.float32)
        cnt_acc[...] = jnp.zeros((G, 1), jnp.float32)

    hg_acc[...] += lax.dot_general(oh, h2, (((0,), (0,)), ((), ())),
                                   preferred_element_type=jnp.float32)
    cnt_acc[...] += lax.dot_general(oh, jnp.ones((_BLK, 1), jnp.float32),
                                    (((0,), (0,)), ((), ())),
                                    preferred_element_type=jnp.float32)

    @pl.when(i == _NBLK - 1)
    def _():
        hg = hg_acc[...] / jnp.maximum(cnt_acc[...], 1.0)
        bninv = 1.0 / jnp.sqrt(1.0 + EPS)

        def lin_t(x, w_ref, b_ref):
            return lax.dot_general(x, w_ref[...], (((1,), (1,)), ((), ())),
                                   preferred_element_type=jnp.float32) + b_ref[...]

        h1 = _leaky(g0_ref[...] * lin_t(hg, we1_ref, be1_ref) * bninv + bb0_ref[...], 0.05)
        hb = _leaky(g1_ref[...] * lin_t(h1, we2_ref, be2_ref) * bninv + bb1_ref[...], 0.05)
        h3 = _leaky(lin_t(hb, we3_ref, be3_ref), 0.1)
        h4 = _leaky(lin_t(h3, we4_ref, be4_ref), 0.1)
        y = lin_t(h4, we5_ref, be5_ref)
        m = jnp.max(y, axis=1, keepdims=True)
        z = y - m
        out_ref[...] = z - jnp.log(jnp.sum(jnp.exp(z), axis=1, keepdims=True))


_final_call = pl.pallas_call(
    _final_body,
    grid=(_NBLK,),
    in_specs=[
        pl.BlockSpec((NCORE, _BLK, H), lambda i: (0, i, 0)),
        pl.BlockSpec((_BLK, 1), lambda i: (i, 0)),
        pl.BlockSpec((_BLK, 1), lambda i: (i, 0)),
        pl.BlockSpec((H, H), lambda i: (0, 0)),
        pl.BlockSpec((1, H), lambda i: (0, 0)),
        pl.BlockSpec((H, H), lambda i: (0, 0)),
        pl.BlockSpec((1, H), lambda i: (0, 0)),
        pl.BlockSpec((1, H), lambda i: (0, 0)),
        pl.BlockSpec((1, H), lambda i: (0, 0)),
        pl.BlockSpec((H, H), lambda i: (0, 0)),
        pl.BlockSpec((1, H), lambda i: (0, 0)),
        pl.BlockSpec((1, H), lambda i: (0, 0)),
        pl.BlockSpec((1, H), lambda i: (0, 0)),
        pl.BlockSpec((G, H), lambda i: (0, 0)),
        pl.BlockSpec((1, G), lambda i: (0, 0)),
        pl.BlockSpec((32, G), lambda i: (0, 0)),
        pl.BlockSpec((1, 32), lambda i: (0, 0)),
        pl.BlockSpec((10, 32), lambda i: (0, 0)),
        pl.BlockSpec((1, 10), lambda i: (0, 0)),
    ],
    out_specs=pl.BlockSpec((G, 10), lambda i: (0, 0)),
    out_shape=jax.ShapeDtypeStruct((G, 10), jnp.float32),
    scratch_shapes=[
        pltpu.VMEM((G, H), jnp.float32),
        pltpu.VMEM((G, 1), jnp.float32),
    ],
)


def kernel(node_feat, edge_index, graph_ids, emb, W1, b1, W2, b2,
           We1, be1, g0, bb0, We2, be2, g1, bb1, We3, be3, We4, be4, We5, be5):
    nf2 = jnp.concatenate(
        [node_feat, jnp.zeros((NPAD - N,), jnp.int32)]).reshape(NW, ROWS_PER_W)
    pad_e = N + jnp.arange(EPAD - E, dtype=jnp.int32) % (NPAD - N)
    src3 = jnp.concatenate([edge_index[0], pad_e]).reshape(NW, NSTEPS, ECHUNK)
    dst3 = jnp.concatenate([edge_index[1], pad_e]).reshape(NW, NSTEPS, ECHUNK)
    gid = jnp.concatenate(
        [graph_ids, jnp.full((NPAD - N,), G, jnp.int32)]).reshape(NPAD, 1)

    h0, odp, idp = _prep_call(nf2, emb, src3, dst3)
    hs0, osc, isc = _scale_call(jnp.transpose(odp), jnp.transpose(idp), h0)

    part1 = _agg_call(hs0, src3, dst3)
    hs1 = _layer1_call(part1, isc, osc, W1, b1.reshape(1, H))

    part2 = _agg_call(hs1, src3, dst3)
    out = _final_call(part2, isc, gid, W2, b2.reshape(1, H),
                      We1, be1.reshape(1, H), g0.reshape(1, H), bb0.reshape(1, H),
                      We2, be2.reshape(1, H), g1.reshape(1, H), bb1.reshape(1, H),
                      We3, be3.reshape(1, G), We4, be4.reshape(1, 32),
                      We5, be5.reshape(1, 10))
    return out


# speedup vs baseline: 3.2306x; 1.0419x over previous
import functools

import jax
import jax.numpy as jnp
from jax import lax
from jax.experimental import pallas as pl
from jax.experimental.pallas import tpu as pltpu
from jax.experimental.pallas import tpu_sc as plsc

N = 10000
E = 320000
H = 128
G = 64
EPS = 1e-5

NCORE = 2
NSUB = 16
NW = NCORE * NSUB
NPAD = 10240
ROWS_PER_W = NPAD // NW
GCHUNKS = ((0, 128), (128, 128), (256, 64))
EPW = 80 * 128
EPAD = NW * EPW
ECHUNK = 128
NSTEPS = EPW // ECHUNK
NBUF = 2
IBLK = 40
NIBLK = NSTEPS // IBLK
DUMMY = 10200
RPT = NPAD // NSUB
DPT = RPT

_mesh = plsc.VectorSubcoreMesh(core_axis_name="c", subcore_axis_name="s")


def _zero_rows(ref, nrows):
    def body(r, _):
        for k in range(H // 16):
            ref[r, pl.ds(k * 16, 16)] = jnp.zeros((16,), jnp.float32)
        return 0
    lax.fori_loop(0, nrows, body, 0)


def _zero_vec(ref, n):
    def body(r, _):
        ref[pl.ds(r * 16, 16)] = jnp.zeros((16,), jnp.float32)
        return 0
    lax.fori_loop(0, n // 16, body, 0)


def _prep_body(nf2, emb, src3, dst3, h0, odp, idp,
               nf_v, src_v, dst_v, rows_v, ones_v, dbuf_v, od_s, id_s, sem, dsem):
    c = lax.axis_index("c")
    s = lax.axis_index("s")
    wid = c * NSUB + s

    pltpu.sync_copy(nf2.at[wid], nf_v)
    pltpu.sync_copy(src3.at[wid], src_v)
    pltpu.sync_copy(dst3.at[wid], dst_v)

    _zero_vec(dbuf_v, DPT)
    pltpu.sync_copy(dbuf_v, od_s.at[pl.ds(s * DPT, DPT)])
    pltpu.sync_copy(dbuf_v, id_s.at[pl.ds(s * DPT, DPT)])
    for k in range(H // 16):
        ones_v[pl.ds(k * 16, 16)] = jnp.full((16,), 1.0, jnp.float32)
    plsc.subcore_barrier()

    def dfire(j, _):
        pltpu.async_copy(ones_v, od_s.at[src_v.at[j]], dsem, add=True)
        pltpu.async_copy(ones_v, id_s.at[dst_v.at[j]], dsem, add=True)
        return 0
    lax.fori_loop(0, NSTEPS, dfire, 0)

    for off, sz in GCHUNKS:
        pltpu.async_copy(emb.at[nf_v.at[pl.ds(off, sz)]],
                         rows_v.at[pl.ds(0, sz)], sem).wait()
        pltpu.sync_copy(rows_v.at[pl.ds(0, sz)],
                        h0.at[pl.ds(wid * ROWS_PER_W + off, sz)])

    def ddrain(j, _):
        pltpu.make_async_copy(ones_v, od_s.at[src_v.at[j]], dsem).wait()
        pltpu.make_async_copy(ones_v, id_s.at[dst_v.at[j]], dsem).wait()
        return 0
    lax.fori_loop(0, NSTEPS, ddrain, 0)

    plsc.subcore_barrier()

    pltpu.sync_copy(od_s.at[pl.ds(s * DPT, DPT)], dbuf_v)
    pltpu.sync_copy(dbuf_v, odp.at[c, pl.ds(s * DPT, DPT)])
    pltpu.sync_copy(id_s.at[pl.ds(s * DPT, DPT)], dbuf_v)
    pltpu.sync_copy(dbuf_v, idp.at[c, pl.ds(s * DPT, DPT)])


_prep_call = pl.kernel(
    _prep_body,
    out_type=(
        jax.ShapeDtypeStruct((NPAD, H), jnp.float32),
        jax.ShapeDtypeStruct((NCORE, NPAD), jnp.float32),
        jax.ShapeDtypeStruct((NCORE, NPAD), jnp.float32),
    ),
    mesh=_mesh,
    scratch_types=[
        pltpu.VMEM((ROWS_PER_W,), jnp.int32),
        pltpu.VMEM((NSTEPS, ECHUNK), jnp.int32),
        pltpu.VMEM((NSTEPS, ECHUNK), jnp.int32),
        pltpu.VMEM((ECHUNK, H), jnp.float32),
        pltpu.VMEM((ECHUNK,), jnp.float32),
        pltpu.VMEM((DPT,), jnp.float32),
        pltpu.VMEM_SHARED((NPAD,), jnp.float32),
        pltpu.VMEM_SHARED((NPAD,), jnp.float32),
        pltpu.SemaphoreType.DMA,
        pltpu.SemaphoreType.DMA,
    ],
)


def _agg_body(hs, src3, dst3, part, src_i, dst_i, rows_v, agg_s, sem):
    c = lax.axis_index("c")
    s = lax.axis_index("s")
    wid = c * NSUB + s

    _zero_rows(rows_v.at[0], ECHUNK)
    for i in range(RPT // ECHUNK):
        pltpu.sync_copy(rows_v.at[0], agg_s.at[pl.ds(s * RPT + i * ECHUNK, ECHUNK)])
    plsc.subcore_barrier()

    for blk in range(NIBLK):
        pltpu.sync_copy(src3.at[wid, pl.ds(blk * IBLK, IBLK)], src_i)
        pltpu.sync_copy(dst3.at[wid, pl.ds(blk * IBLK, IBLK)], dst_i)
        for b in range(NBUF):
            pltpu.async_copy(hs.at[src_i.at[b]], rows_v.at[b], sem)

        def outer(g, _):
            for b in range(NBUF):
                j = g * NBUF + b
                pltpu.make_async_copy(hs.at[src_i.at[j]], rows_v.at[b], sem).wait()
                pltpu.sync_copy(rows_v.at[b], agg_s.at[dst_i.at[j]], add=True)

                @pl.when(j + NBUF < IBLK)
                def _():
                    pltpu.async_copy(hs.at[src_i.at[j + NBUF]], rows_v.at[b], sem)
            return 0
        lax.fori_loop(0, IBLK // NBUF, outer, 0)

    plsc.subcore_barrier()

    pltpu.sync_copy(agg_s.at[pl.ds(s * RPT, RPT)],
                    part.at[c, pl.ds(s * RPT, RPT)])


_agg_call = pl.kernel(
    _agg_body,
    out_type=jax.ShapeDtypeStruct((NCORE, NPAD, H), jnp.float32),
    mesh=_mesh,
    scratch_types=[
        pltpu.VMEM((IBLK, ECHUNK), jnp.int32),
        pltpu.VMEM((IBLK, ECHUNK), jnp.int32),
        pltpu.VMEM((NBUF, ECHUNK, H), jnp.float32),
        pltpu.VMEM_SHARED((NPAD, H), jnp.float32),
        pltpu.SemaphoreType.DMA,
    ],
)


_BLK = 1024
_NBLK = NPAD // _BLK


def _pre1_body(odp_ref, idp_ref, h0_ref, w_ref, m_ref, osc_ref, isc_ref):
    od = odp_ref[...]
    idg = idp_ref[...]
    osc = lax.rsqrt(jnp.maximum(od[:, 0:1] + od[:, 1:2], 1.0))
    isc = lax.rsqrt(jnp.maximum(idg[:, 0:1] + idg[:, 1:2], 1.0))
    m_ref[...] = lax.dot_general(h0_ref[...] * osc, w_ref[...],
                                 (((1,), (0,)), ((), ())),
                                 preferred_element_type=jnp.float32)
    osc_ref[...] = osc
    isc_ref[...] = isc


_pre1_call = pl.pallas_call(
    _pre1_body,
    grid=(_NBLK,),
    in_specs=[
        pl.BlockSpec((_BLK, 2), lambda i: (i, 0)),
        pl.BlockSpec((_BLK, 2), lambda i: (i, 0)),
        pl.BlockSpec((_BLK, H), lambda i: (i, 0)),
        pl.BlockSpec((H, H), lambda i: (0, 0)),
    ],
    out_specs=[
        pl.BlockSpec((_BLK, H), lambda i: (i, 0)),
        pl.BlockSpec((_BLK, 1), lambda i: (i, 0)),
        pl.BlockSpec((_BLK, 1), lambda i: (i, 0)),
    ],
    out_shape=[
        jax.ShapeDtypeStruct((NPAD, H), jnp.float32),
        jax.ShapeDtypeStruct((NPAD, 1), jnp.float32),
        jax.ShapeDtypeStruct((NPAD, 1), jnp.float32),
    ],
)


def _mid_body(part_ref, isc_ref, osc_ref, b_ref, w_ref, out_ref):
    p = part_ref[0] + part_ref[1]
    t = jnp.maximum(p * isc_ref[...] + b_ref[...], 0.0)
    out_ref[...] = lax.dot_general(t * osc_ref[...], w_ref[...],
                                   (((1,), (0,)), ((), ())),
                                   preferred_element_type=jnp.float32)


_mid_call = pl.pallas_call(
    _mid_body,
    grid=(_NBLK,),
    in_specs=[
        pl.BlockSpec((NCORE, _BLK, H), lambda i: (0, i, 0)),
        pl.BlockSpec((_BLK, 1), lambda i: (i, 0)),
        pl.BlockSpec((_BLK, 1), lambda i: (i, 0)),
        pl.BlockSpec((1, H), lambda i: (0, 0)),
        pl.BlockSpec((H, H), lambda i: (0, 0)),
    ],
    out_specs=pl.BlockSpec((_BLK, H), lambda i: (i, 0)),
    out_shape=jax.ShapeDtypeStruct((NPAD, H), jnp.float32),
)


def _leaky(x, slope):
    return jnp.where(x >= 0, x, slope * x)


def _final_body(part_ref, isc_ref, gid_ref, b2_ref,
                we1_ref, be1_ref, g0_ref, bb0_ref,
                we2_ref, be2_ref, g1_ref, bb1_ref,
                we3_ref, be3_ref, we4_ref, be4_ref, we5_ref, be5_ref,
                out_ref, hg_acc, cnt_acc):
    i = pl.program_id(0)

    p = part_ref[0] + part_ref[1]
    h2 = jnp.maximum(p * isc_ref[...] + b2_ref[...], 0.0)

    ids = gid_ref[...]
    seg = lax.broadcasted_iota(jnp.int32, (_BLK, G), 1)
    oh = (ids == seg).astype(jnp.float32)

    @pl.when(i == 0)
    def _():
        hg_acc[...] = jnp.zeros((G, H), jnp.float32)
        cnt_acc[...] = jnp.zeros((G, 1), jnp.float32)

    hg_acc[...] += lax.dot_general(oh, h2, (((0,), (0,)), ((), ())),
                                   preferred_element_type=jnp.float32)
    cnt_acc[...] += lax.dot_general(oh, jnp.ones((_BLK, 1), jnp.float32),
                                    (((0,), (0,)), ((), ())),
                                    preferred_element_type=jnp.float32)

    @pl.when(i == _NBLK - 1)
    def _():
        hg = hg_acc[...] / jnp.maximum(cnt_acc[...], 1.0)
        bninv = 1.0 / jnp.sqrt(1.0 + EPS)

        def lin_t(x, w_ref, b_ref):
            return lax.dot_general(x, w_ref[...], (((1,), (1,)), ((), ())),
                                   preferred_element_type=jnp.float32) + b_ref[...]

        h1 = _leaky(g0_ref[...] * lin_t(hg, we1_ref, be1_ref) * bninv + bb0_ref[...], 0.05)
        hb = _leaky(g1_ref[...] * lin_t(h1, we2_ref, be2_ref) * bninv + bb1_ref[...], 0.05)
        h3 = _leaky(lin_t(hb, we3_ref, be3_ref), 0.1)
        h4 = _leaky(lin_t(h3, we4_ref, be4_ref), 0.1)
        y = lin_t(h4, we5_ref, be5_ref)
        m = jnp.max(y, axis=1, keepdims=True)
        z = y - m
        out_ref[...] = z - jnp.log(jnp.sum(jnp.exp(z), axis=1, keepdims=True))


_final_call = pl.pallas_call(
    _final_body,
    grid=(_NBLK,),
    in_specs=[
        pl.BlockSpec((NCORE, _BLK, H), lambda i: (0, i, 0)),
        pl.BlockSpec((_BLK, 1), lambda i: (i, 0)),
        pl.BlockSpec((_BLK, 1), lambda i: (i, 0)),
        pl.BlockSpec((1, H), lambda i: (0, 0)),
        pl.BlockSpec((H, H), lambda i: (0, 0)),
        pl.BlockSpec((1, H), lambda i: (0, 0)),
        pl.BlockSpec((1, H), lambda i: (0, 0)),
        pl.BlockSpec((1, H), lambda i: (0, 0)),
        pl.BlockSpec((H, H), lambda i: (0, 0)),
        pl.BlockSpec((1, H), lambda i: (0, 0)),
        pl.BlockSpec((1, H), lambda i: (0, 0)),
        pl.BlockSpec((1, H), lambda i: (0, 0)),
        pl.BlockSpec((G, H), lambda i: (0, 0)),
        pl.BlockSpec((1, G), lambda i: (0, 0)),
        pl.BlockSpec((32, G), lambda i: (0, 0)),
        pl.BlockSpec((1, 32), lambda i: (0, 0)),
        pl.BlockSpec((10, 32), lambda i: (0, 0)),
        pl.BlockSpec((1, 10), lambda i: (0, 0)),
    ],
    out_specs=pl.BlockSpec((G, 10), lambda i: (0, 0)),
    out_shape=jax.ShapeDtypeStruct((G, 10), jnp.float32),
    scratch_shapes=[
        pltpu.VMEM((G, H), jnp.float32),
        pltpu.VMEM((G, 1), jnp.float32),
    ],
)


def kernel(node_feat, edge_index, graph_ids, emb, W1, b1, W2, b2,
           We1, be1, g0, bb0, We2, be2, g1, bb1, We3, be3, We4, be4, We5, be5):
    nf2 = jnp.concatenate(
        [node_feat, jnp.zeros((NPAD - N,), jnp.int32)]).reshape(NW, ROWS_PER_W)
    pad_e = N + jnp.arange(EPAD - E, dtype=jnp.int32) % (NPAD - N)
    src3 = jnp.concatenate([edge_index[0], pad_e]).reshape(NW, NSTEPS, ECHUNK)
    dst3 = jnp.concatenate([edge_index[1], pad_e]).reshape(NW, NSTEPS, ECHUNK)
    gid = jnp.concatenate(
        [graph_ids, jnp.full((NPAD - N,), G, jnp.int32)]).reshape(NPAD, 1)

    h0, odp, idp = _prep_call(nf2, emb, src3, dst3)
    m0, osc, isc = _pre1_call(jnp.transpose(odp), jnp.transpose(idp), h0, W1)

    part1 = _agg_call(m0, src3, dst3)
    m1 = _mid_call(part1, isc, osc, b1.reshape(1, H), W2)

    part2 = _agg_call(m1, src3, dst3)
    out = _final_call(part2, isc, gid, b2.reshape(1, H),
                      We1, be1.reshape(1, H), g0.reshape(1, H), bb0.reshape(1, H),
                      We2, be2.reshape(1, H), g1.reshape(1, H), bb1.reshape(1, H),
                      We3, be3.reshape(1, G), We4, be4.reshape(1, 32),
                      We5, be5.reshape(1, 10))
    return out


# speedup vs baseline: 3.2837x; 1.0164x over previous
import functools

import jax
import jax.numpy as jnp
from jax import lax
from jax.experimental import pallas as pl
from jax.experimental.pallas import tpu as pltpu
from jax.experimental.pallas import tpu_sc as plsc

N = 10000
E = 320000
H = 128
G = 64
EPS = 1e-5

NCORE = 2
NSUB = 16
NW = NCORE * NSUB
NPAD = 10240
ROWS_PER_W = NPAD // NW
GCHUNKS = ((0, 128), (128, 128), (256, 64))
EPW = 80 * 128
EPAD = NW * EPW
ECHUNK = 128
NSTEPS = EPW // ECHUNK
NBUF = 2
IBLK = 40
NIBLK = NSTEPS // IBLK
DUMMY = 10200
RPT = NPAD // NSUB
DPT = RPT

_mesh = plsc.VectorSubcoreMesh(core_axis_name="c", subcore_axis_name="s")


def _zero_rows(ref, nrows):
    def body(r, _):
        for k in range(H // 16):
            ref[r, pl.ds(k * 16, 16)] = jnp.zeros((16,), jnp.float32)
        return 0
    lax.fori_loop(0, nrows, body, 0)


def _zero_vec(ref, n):
    def body(r, _):
        ref[pl.ds(r * 16, 16)] = jnp.zeros((16,), jnp.float32)
        return 0
    lax.fori_loop(0, n // 16, body, 0)


def _prep_body(nf2, emb, src3, dst3, h0, odp, idp,
               nf_v, src_v, dst_v, rows_v, ones_v, dbuf_v, od_s, id_s, sem, dsem):
    c = lax.axis_index("c")
    s = lax.axis_index("s")
    wid = c * NSUB + s

    pltpu.sync_copy(nf2.at[wid], nf_v)
    pltpu.sync_copy(src3.at[wid], src_v)
    pltpu.sync_copy(dst3.at[wid], dst_v)

    _zero_vec(dbuf_v, DPT)
    pltpu.sync_copy(dbuf_v, od_s.at[pl.ds(s * DPT, DPT)])
    pltpu.sync_copy(dbuf_v, id_s.at[pl.ds(s * DPT, DPT)])
    for k in range(H // 16):
        ones_v[pl.ds(k * 16, 16)] = jnp.full((16,), 1.0, jnp.float32)
    plsc.subcore_barrier()

    def dfire(j, _):
        pltpu.async_copy(ones_v, od_s.at[src_v.at[j]], dsem, add=True)
        pltpu.async_copy(ones_v, id_s.at[dst_v.at[j]], dsem, add=True)
        return 0
    lax.fori_loop(0, NSTEPS, dfire, 0)

    for off, sz in GCHUNKS:
        pltpu.async_copy(emb.at[nf_v.at[pl.ds(off, sz)]],
                         rows_v.at[pl.ds(0, sz)], sem).wait()
        pltpu.sync_copy(rows_v.at[pl.ds(0, sz)],
                        h0.at[pl.ds(wid * ROWS_PER_W + off, sz)])

    def ddrain(j, _):
        pltpu.make_async_copy(ones_v, od_s.at[src_v.at[j]], dsem).wait()
        pltpu.make_async_copy(ones_v, id_s.at[dst_v.at[j]], dsem).wait()
        return 0
    lax.fori_loop(0, NSTEPS, ddrain, 0)

    plsc.subcore_barrier()

    pltpu.sync_copy(od_s.at[pl.ds(s * DPT, DPT)], odp.at[c, pl.ds(s * DPT, DPT)])
    pltpu.sync_copy(id_s.at[pl.ds(s * DPT, DPT)], idp.at[c, pl.ds(s * DPT, DPT)])


_prep_call = pl.kernel(
    _prep_body,
    out_type=(
        jax.ShapeDtypeStruct((NPAD, H), jnp.float32),
        jax.ShapeDtypeStruct((NCORE, NPAD), jnp.float32),
        jax.ShapeDtypeStruct((NCORE, NPAD), jnp.float32),
    ),
    mesh=_mesh,
    scratch_types=[
        pltpu.VMEM((ROWS_PER_W,), jnp.int32),
        pltpu.VMEM((NSTEPS, ECHUNK), jnp.int32),
        pltpu.VMEM((NSTEPS, ECHUNK), jnp.int32),
        pltpu.VMEM((ECHUNK, H), jnp.float32),
        pltpu.VMEM((ECHUNK,), jnp.float32),
        pltpu.VMEM((DPT,), jnp.float32),
        pltpu.VMEM_SHARED((NPAD,), jnp.float32),
        pltpu.VMEM_SHARED((NPAD,), jnp.float32),
        pltpu.SemaphoreType.DMA,
        pltpu.SemaphoreType.DMA,
    ],
)


def _agg_body(hs, src3, dst3, part, src_i, dst_i, rows_v, agg_s, sem):
    c = lax.axis_index("c")
    s = lax.axis_index("s")
    wid = c * NSUB + s

    _zero_rows(rows_v.at[0], ECHUNK)
    for i in range(RPT // ECHUNK):
        pltpu.sync_copy(rows_v.at[0], agg_s.at[pl.ds(s * RPT + i * ECHUNK, ECHUNK)])
    plsc.subcore_barrier()

    for blk in range(NIBLK):
        pltpu.sync_copy(src3.at[wid, pl.ds(blk * IBLK, IBLK)], src_i)
        pltpu.sync_copy(dst3.at[wid, pl.ds(blk * IBLK, IBLK)], dst_i)
        for b in range(NBUF):
            pltpu.async_copy(hs.at[src_i.at[b]], rows_v.at[b], sem)

        def outer(g, _):
            for b in range(NBUF):
                j = g * NBUF + b
                pltpu.make_async_copy(hs.at[src_i.at[j]], rows_v.at[b], sem).wait()
                pltpu.sync_copy(rows_v.at[b], agg_s.at[dst_i.at[j]], add=True)

                @pl.when(j + NBUF < IBLK)
                def _():
                    pltpu.async_copy(hs.at[src_i.at[j + NBUF]], rows_v.at[b], sem)
            return 0
        lax.fori_loop(0, IBLK // NBUF, outer, 0)

    plsc.subcore_barrier()

    pltpu.sync_copy(agg_s.at[pl.ds(s * RPT, RPT)],
                    part.at[c, pl.ds(s * RPT, RPT)])


_agg_call = pl.kernel(
    _agg_body,
    out_type=jax.ShapeDtypeStruct((NCORE, NPAD, H), jnp.float32),
    mesh=_mesh,
    scratch_types=[
        pltpu.VMEM((IBLK, ECHUNK), jnp.int32),
        pltpu.VMEM((IBLK, ECHUNK), jnp.int32),
        pltpu.VMEM((NBUF, ECHUNK, H), jnp.float32),
        pltpu.VMEM_SHARED((NPAD, H), jnp.float32),
        pltpu.SemaphoreType.DMA,
    ],
)


_BLK = 2048
_NBLK = NPAD // _BLK


def _pre1_body(odp_ref, idp_ref, h0_ref, w_ref, m_ref, osc_ref, isc_ref):
    od = odp_ref[...]
    idg = idp_ref[...]
    osc = lax.rsqrt(jnp.maximum(od[:, 0:1] + od[:, 1:2], 1.0))
    isc = lax.rsqrt(jnp.maximum(idg[:, 0:1] + idg[:, 1:2], 1.0))
    m_ref[...] = lax.dot_general(h0_ref[...] * osc, w_ref[...],
                                 (((1,), (0,)), ((), ())),
                                 preferred_element_type=jnp.float32)
    osc_ref[...] = osc
    isc_ref[...] = isc


_pre1_call = pl.pallas_call(
    _pre1_body,
    grid=(_NBLK,),
    in_specs=[
        pl.BlockSpec((_BLK, 2), lambda i: (i, 0)),
        pl.BlockSpec((_BLK, 2), lambda i: (i, 0)),
        pl.BlockSpec((_BLK, H), lambda i: (i, 0)),
        pl.BlockSpec((H, H), lambda i: (0, 0)),
    ],
    out_specs=[
        pl.BlockSpec((_BLK, H), lambda i: (i, 0)),
        pl.BlockSpec((_BLK, 1), lambda i: (i, 0)),
        pl.BlockSpec((_BLK, 1), lambda i: (i, 0)),
    ],
    out_shape=[
        jax.ShapeDtypeStruct((NPAD, H), jnp.float32),
        jax.ShapeDtypeStruct((NPAD, 1), jnp.float32),
        jax.ShapeDtypeStruct((NPAD, 1), jnp.float32),
    ],
)


def _mid_body(part_ref, isc_ref, osc_ref, b_ref, w_ref, out_ref):
    p = part_ref[0] + part_ref[1]
    t = jnp.maximum(p * isc_ref[...] + b_ref[...], 0.0)
    out_ref[...] = lax.dot_general(t * osc_ref[...], w_ref[...],
                                   (((1,), (0,)), ((), ())),
                                   preferred_element_type=jnp.float32)


_mid_call = pl.pallas_call(
    _mid_body,
    grid=(_NBLK,),
    in_specs=[
        pl.BlockSpec((NCORE, _BLK, H), lambda i: (0, i, 0)),
        pl.BlockSpec((_BLK, 1), lambda i: (i, 0)),
        pl.BlockSpec((_BLK, 1), lambda i: (i, 0)),
        pl.BlockSpec((1, H), lambda i: (0, 0)),
        pl.BlockSpec((H, H), lambda i: (0, 0)),
    ],
    out_specs=pl.BlockSpec((_BLK, H), lambda i: (i, 0)),
    out_shape=jax.ShapeDtypeStruct((NPAD, H), jnp.float32),
)


def _leaky(x, slope):
    return jnp.where(x >= 0, x, slope * x)


def _final_body(part_ref, isc_ref, gid_ref, b2_ref,
                we1_ref, be1_ref, g0_ref, bb0_ref,
                we2_ref, be2_ref, g1_ref, bb1_ref,
                we3_ref, be3_ref, we4_ref, be4_ref, we5_ref, be5_ref,
                out_ref, hg_acc, cnt_acc):
    i = pl.program_id(0)

    p = part_ref[0] + part_ref[1]
    h2 = jnp.maximum(p * isc_ref[...] + b2_ref[...], 0.0)

    ids = gid_ref[...]
    seg = lax.broadcasted_iota(jnp.int32, (_BLK, G), 1)
    oh = (ids == seg).astype(jnp.float32)

    @pl.when(i == 0)
    def _():
        hg_acc[...] = jnp.zeros((G, H), jnp.float32)
        cnt_acc[...] = jnp.zeros((G, 1), jnp.float32)

    hg_acc[...] += lax.dot_general(oh, h2, (((0,), (0,)), ((), ())),
                                   preferred_element_type=jnp.float32)
    cnt_acc[...] += lax.dot_general(oh, jnp.ones((_BLK, 1), jnp.float32),
                                    (((0,), (0,)), ((), ())),
                                    preferred_element_type=jnp.float32)

    @pl.when(i == _NBLK - 1)
    def _():
        hg = hg_acc[...] / jnp.maximum(cnt_acc[...], 1.0)
        bninv = 1.0 / jnp.sqrt(1.0 + EPS)

        def lin_t(x, w_ref, b_ref):
            return lax.dot_general(x, w_ref[...], (((1,), (1,)), ((), ())),
                                   preferred_element_type=jnp.float32) + b_ref[...]

        h1 = _leaky(g0_ref[...] * lin_t(hg, we1_ref, be1_ref) * bninv + bb0_ref[...], 0.05)
        hb = _leaky(g1_ref[...] * lin_t(h1, we2_ref, be2_ref) * bninv + bb1_ref[...], 0.05)
        h3 = _leaky(lin_t(hb, we3_ref, be3_ref), 0.1)
        h4 = _leaky(lin_t(h3, we4_ref, be4_ref), 0.1)
        y = lin_t(h4, we5_ref, be5_ref)
        m = jnp.max(y, axis=1, keepdims=True)
        z = y - m
        out_ref[...] = z - jnp.log(jnp.sum(jnp.exp(z), axis=1, keepdims=True))


_final_call = pl.pallas_call(
    _final_body,
    grid=(_NBLK,),
    in_specs=[
        pl.BlockSpec((NCORE, _BLK, H), lambda i: (0, i, 0)),
        pl.BlockSpec((_BLK, 1), lambda i: (i, 0)),
        pl.BlockSpec((_BLK, 1), lambda i: (i, 0)),
        pl.BlockSpec((1, H), lambda i: (0, 0)),
        pl.BlockSpec((H, H), lambda i: (0, 0)),
        pl.BlockSpec((1, H), lambda i: (0, 0)),
        pl.BlockSpec((1, H), lambda i: (0, 0)),
        pl.BlockSpec((1, H), lambda i: (0, 0)),
        pl.BlockSpec((H, H), lambda i: (0, 0)),
        pl.BlockSpec((1, H), lambda i: (0, 0)),
        pl.BlockSpec((1, H), lambda i: (0, 0)),
        pl.BlockSpec((1, H), lambda i: (0, 0)),
        pl.BlockSpec((G, H), lambda i: (0, 0)),
        pl.BlockSpec((1, G), lambda i: (0, 0)),
        pl.BlockSpec((32, G), lambda i: (0, 0)),
        pl.BlockSpec((1, 32), lambda i: (0, 0)),
        pl.BlockSpec((10, 32), lambda i: (0, 0)),
        pl.BlockSpec((1, 10), lambda i: (0, 0)),
    ],
    out_specs=pl.BlockSpec((G, 10), lambda i: (0, 0)),
    out_shape=jax.ShapeDtypeStruct((G, 10), jnp.float32),
    scratch_shapes=[
        pltpu.VMEM((G, H), jnp.float32),
        pltpu.VMEM((G, 1), jnp.float32),
    ],
)


def kernel(node_feat, edge_index, graph_ids, emb, W1, b1, W2, b2,
           We1, be1, g0, bb0, We2, be2, g1, bb1, We3, be3, We4, be4, We5, be5):
    nf2 = jnp.concatenate(
        [node_feat, jnp.zeros((NPAD - N,), jnp.int32)]).reshape(NW, ROWS_PER_W)
    pad_e = N + jnp.arange(EPAD - E, dtype=jnp.int32) % (NPAD - N)
    src3 = jnp.concatenate([edge_index[0], pad_e]).reshape(NW, NSTEPS, ECHUNK)
    dst3 = jnp.concatenate([edge_index[1], pad_e]).reshape(NW, NSTEPS, ECHUNK)
    gid = jnp.concatenate(
        [graph_ids, jnp.full((NPAD - N,), G, jnp.int32)]).reshape(NPAD, 1)

    h0, odp, idp = _prep_call(nf2, emb, src3, dst3)
    m0, osc, isc = _pre1_call(jnp.transpose(odp), jnp.transpose(idp), h0, W1)

    part1 = _agg_call(m0, src3, dst3)
    m1 = _mid_call(part1, isc, osc, b1.reshape(1, H), W2)

    part2 = _agg_call(m1, src3, dst3)
    out = _final_call(part2, isc, gid, b2.reshape(1, H),
                      We1, be1.reshape(1, H), g0.reshape(1, H), bb0.reshape(1, H),
                      We2, be2.reshape(1, H), g1.reshape(1, H), bb1.reshape(1, H),
                      We3, be3.reshape(1, G), We4, be4.reshape(1, 32),
                      We5, be5.reshape(1, 10))
    return out


# speedup vs baseline: 3.4027x; 1.0362x over previous
import functools

import jax
import jax.numpy as jnp
from jax import lax
from jax.experimental import pallas as pl
from jax.experimental.pallas import tpu as pltpu
from jax.experimental.pallas import tpu_sc as plsc

N = 10000
E = 320000
H = 128
G = 64
EPS = 1e-5

NCORE = 2
NSUB = 16
NW = NCORE * NSUB
NPAD = 10240
ROWS_PER_W = NPAD // NW
GCHUNKS = ((0, 128), (128, 128), (256, 64))
EPW = 10240
EPAD = NW * EPW
ECHUNK = 64
NSTEPS = EPW // ECHUNK
NBUF = 4
IBLK = 40
NIBLK = NSTEPS // IBLK
PCHUNK = 128
DUMMY = 10200
RPT = NPAD // NSUB
DPT = RPT

_mesh = plsc.VectorSubcoreMesh(core_axis_name="c", subcore_axis_name="s")


def _zero_rows(ref, nrows):
    def body(r, _):
        for k in range(H // 16):
            ref[r, pl.ds(k * 16, 16)] = jnp.zeros((16,), jnp.float32)
        return 0
    lax.fori_loop(0, nrows, body, 0)


def _zero_vec(ref, n):
    def body(r, _):
        ref[pl.ds(r * 16, 16)] = jnp.zeros((16,), jnp.float32)
        return 0
    lax.fori_loop(0, n // 16, body, 0)


def _prep_body(nf2, emb, src3, dst3, h0, odp, idp,
               nf_v, src_v, dst_v, rows_v, ones_v, dbuf_v, od_s, id_s, sem, dsem):
    c = lax.axis_index("c")
    s = lax.axis_index("s")
    wid = c * NSUB + s

    pltpu.sync_copy(nf2.at[wid], nf_v)
    pltpu.sync_copy(src3.at[wid], src_v)
    pltpu.sync_copy(dst3.at[wid], dst_v)

    _zero_vec(dbuf_v, DPT)
    pltpu.sync_copy(dbuf_v, od_s.at[pl.ds(s * DPT, DPT)])
    pltpu.sync_copy(dbuf_v, id_s.at[pl.ds(s * DPT, DPT)])
    for k in range(ECHUNK // 16):
        ones_v[pl.ds(k * 16, 16)] = jnp.full((16,), 1.0, jnp.float32)
    plsc.subcore_barrier()

    def dfire(j, _):
        pltpu.async_copy(ones_v, od_s.at[src_v.at[j]], dsem, add=True)
        pltpu.async_copy(ones_v, id_s.at[dst_v.at[j]], dsem, add=True)
        return 0
    lax.fori_loop(0, NSTEPS, dfire, 0)

    for off, sz in GCHUNKS:
        pltpu.async_copy(emb.at[nf_v.at[pl.ds(off, sz)]],
                         rows_v.at[pl.ds(0, sz)], sem).wait()
        pltpu.sync_copy(rows_v.at[pl.ds(0, sz)],
                        h0.at[pl.ds(wid * ROWS_PER_W + off, sz)])

    def ddrain(j, _):
        pltpu.make_async_copy(ones_v, od_s.at[src_v.at[j]], dsem).wait()
        pltpu.make_async_copy(ones_v, id_s.at[dst_v.at[j]], dsem).wait()
        return 0
    lax.fori_loop(0, NSTEPS, ddrain, 0)

    plsc.subcore_barrier()

    pltpu.sync_copy(od_s.at[pl.ds(s * DPT, DPT)], odp.at[c, pl.ds(s * DPT, DPT)])
    pltpu.sync_copy(id_s.at[pl.ds(s * DPT, DPT)], idp.at[c, pl.ds(s * DPT, DPT)])


_prep_call = pl.kernel(
    _prep_body,
    out_type=(
        jax.ShapeDtypeStruct((NPAD, H), jnp.float32),
        jax.ShapeDtypeStruct((NCORE, NPAD), jnp.float32),
        jax.ShapeDtypeStruct((NCORE, NPAD), jnp.float32),
    ),
    mesh=_mesh,
    scratch_types=[
        pltpu.VMEM((ROWS_PER_W,), jnp.int32),
        pltpu.VMEM((NSTEPS, ECHUNK), jnp.int32),
        pltpu.VMEM((NSTEPS, ECHUNK), jnp.int32),
        pltpu.VMEM((PCHUNK, H), jnp.float32),
        pltpu.VMEM((ECHUNK,), jnp.float32),
        pltpu.VMEM((DPT,), jnp.float32),
        pltpu.VMEM_SHARED((NPAD,), jnp.float32),
        pltpu.VMEM_SHARED((NPAD,), jnp.float32),
        pltpu.SemaphoreType.DMA,
        pltpu.SemaphoreType.DMA,
    ],
)


def _agg_body(hs, src3, dst3, part, src_i, dst_i, rows_v, agg_s, sem):
    c = lax.axis_index("c")
    s = lax.axis_index("s")
    wid = c * NSUB + s

    _zero_rows(rows_v.at[0], ECHUNK)
    for i in range(RPT // ECHUNK):
        pltpu.sync_copy(rows_v.at[0], agg_s.at[pl.ds(s * RPT + i * ECHUNK, ECHUNK)])
    plsc.subcore_barrier()

    for blk in range(NIBLK):
        pltpu.sync_copy(src3.at[wid, pl.ds(blk * IBLK, IBLK)], src_i)
        pltpu.sync_copy(dst3.at[wid, pl.ds(blk * IBLK, IBLK)], dst_i)
        for b in range(NBUF):
            pltpu.async_copy(hs.at[src_i.at[b]], rows_v.at[b], sem)

        def outer(g, _):
            for b in range(NBUF):
                j = g * NBUF + b
                pltpu.make_async_copy(hs.at[src_i.at[j]], rows_v.at[b], sem).wait()
                pltpu.sync_copy(rows_v.at[b], agg_s.at[dst_i.at[j]], add=True)

                @pl.when(j + NBUF < IBLK)
                def _():
                    pltpu.async_copy(hs.at[src_i.at[j + NBUF]], rows_v.at[b], sem)
            return 0
        lax.fori_loop(0, IBLK // NBUF, outer, 0)

    plsc.subcore_barrier()

    pltpu.sync_copy(agg_s.at[pl.ds(s * RPT, RPT)],
                    part.at[c, pl.ds(s * RPT, RPT)])


_agg_call = pl.kernel(
    _agg_body,
    out_type=jax.ShapeDtypeStruct((NCORE, NPAD, H), jnp.float32),
    mesh=_mesh,
    scratch_types=[
        pltpu.VMEM((IBLK, ECHUNK), jnp.int32),
        pltpu.VMEM((IBLK, ECHUNK), jnp.int32),
        pltpu.VMEM((NBUF, ECHUNK, H), jnp.float32),
        pltpu.VMEM_SHARED((NPAD, H), jnp.float32),
        pltpu.SemaphoreType.DMA,
    ],
)


_BLK = 2048
_NBLK = NPAD // _BLK


def _pre1_body(odp_ref, idp_ref, h0_ref, w_ref, m_ref, osc_ref, isc_ref):
    od = odp_ref[...]
    idg = idp_ref[...]
    osc = lax.rsqrt(jnp.maximum(od[:, 0:1] + od[:, 1:2], 1.0))
    isc = lax.rsqrt(jnp.maximum(idg[:, 0:1] + idg[:, 1:2], 1.0))
    m_ref[...] = lax.dot_general(h0_ref[...] * osc, w_ref[...],
                                 (((1,), (0,)), ((), ())),
                                 preferred_element_type=jnp.float32)
    osc_ref[...] = osc
    isc_ref[...] = isc


_pre1_call = pl.pallas_call(
    _pre1_body,
    grid=(_NBLK,),
    in_specs=[
        pl.BlockSpec((_BLK, 2), lambda i: (i, 0)),
        pl.BlockSpec((_BLK, 2), lambda i: (i, 0)),
        pl.BlockSpec((_BLK, H), lambda i: (i, 0)),
        pl.BlockSpec((H, H), lambda i: (0, 0)),
    ],
    out_specs=[
        pl.BlockSpec((_BLK, H), lambda i: (i, 0)),
        pl.BlockSpec((_BLK, 1), lambda i: (i, 0)),
        pl.BlockSpec((_BLK, 1), lambda i: (i, 0)),
    ],
    out_shape=[
        jax.ShapeDtypeStruct((NPAD, H), jnp.float32),
        jax.ShapeDtypeStruct((NPAD, 1), jnp.float32),
        jax.ShapeDtypeStruct((NPAD, 1), jnp.float32),
    ],
)


def _mid_body(part_ref, isc_ref, osc_ref, b_ref, w_ref, out_ref):
    p = part_ref[0] + part_ref[1]
    t = jnp.maximum(p * isc_ref[...] + b_ref[...], 0.0)
    out_ref[...] = lax.dot_general(t * osc_ref[...], w_ref[...],
                                   (((1,), (0,)), ((), ())),
                                   preferred_element_type=jnp.float32)


_mid_call = pl.pallas_call(
    _mid_body,
    grid=(_NBLK,),
    in_specs=[
        pl.BlockSpec((NCORE, _BLK, H), lambda i: (0, i, 0)),
        pl.BlockSpec((_BLK, 1), lambda i: (i, 0)),
        pl.BlockSpec((_BLK, 1), lambda i: (i, 0)),
        pl.BlockSpec((1, H), lambda i: (0, 0)),
        pl.BlockSpec((H, H), lambda i: (0, 0)),
    ],
    out_specs=pl.BlockSpec((_BLK, H), lambda i: (i, 0)),
    out_shape=jax.ShapeDtypeStruct((NPAD, H), jnp.float32),
)


def _leaky(x, slope):
    return jnp.where(x >= 0, x, slope * x)


def _final_body(part_ref, isc_ref, gid_ref, b2_ref,
                we1_ref, be1_ref, g0_ref, bb0_ref,
                we2_ref, be2_ref, g1_ref, bb1_ref,
                we3_ref, be3_ref, we4_ref, be4_ref, we5_ref, be5_ref,
                out_ref, hg_acc, cnt_acc):
    i = pl.program_id(0)

    p = part_ref[0] + part_ref[1]
    h2 = jnp.maximum(p * isc_ref[...] + b2_ref[...], 0.0)

    ids = gid_ref[...]
    seg = lax.broadcasted_iota(jnp.int32, (_BLK, G), 1)
    oh = (ids == seg).astype(jnp.float32)

    @pl.when(i == 0)
    def _():
        hg_acc[...] = jnp.zeros((G, H), jnp.float32)
        cnt_acc[...] = jnp.zeros((G, 1), jnp.float32)

    hg_acc[...] += lax.dot_general(oh, h2, (((0,), (0,)), ((), ())),
                                   preferred_element_type=jnp.float32)
    cnt_acc[...] += lax.dot_general(oh, jnp.ones((_BLK, 1), jnp.float32),
                                    (((0,), (0,)), ((), ())),
                                    preferred_element_type=jnp.float32)

    @pl.when(i == _NBLK - 1)
    def _():
        hg = hg_acc[...] / jnp.maximum(cnt_acc[...], 1.0)
        bninv = 1.0 / jnp.sqrt(1.0 + EPS)

        def lin_t(x, w_ref, b_ref):
            return lax.dot_general(x, w_ref[...], (((1,), (1,)), ((), ())),
                                   preferred_element_type=jnp.float32) + b_ref[...]

        h1 = _leaky(g0_ref[...] * lin_t(hg, we1_ref, be1_ref) * bninv + bb0_ref[...], 0.05)
        hb = _leaky(g1_ref[...] * lin_t(h1, we2_ref, be2_ref) * bninv + bb1_ref[...], 0.05)
        h3 = _leaky(lin_t(hb, we3_ref, be3_ref), 0.1)
        h4 = _leaky(lin_t(h3, we4_ref, be4_ref), 0.1)
        y = lin_t(h4, we5_ref, be5_ref)
        m = jnp.max(y, axis=1, keepdims=True)
        z = y - m
        out_ref[...] = z - jnp.log(jnp.sum(jnp.exp(z), axis=1, keepdims=True))


_final_call = pl.pallas_call(
    _final_body,
    grid=(_NBLK,),
    in_specs=[
        pl.BlockSpec((NCORE, _BLK, H), lambda i: (0, i, 0)),
        pl.BlockSpec((_BLK, 1), lambda i: (i, 0)),
        pl.BlockSpec((_BLK, 1), lambda i: (i, 0)),
        pl.BlockSpec((1, H), lambda i: (0, 0)),
        pl.BlockSpec((H, H), lambda i: (0, 0)),
        pl.BlockSpec((1, H), lambda i: (0, 0)),
        pl.BlockSpec((1, H), lambda i: (0, 0)),
        pl.BlockSpec((1, H), lambda i: (0, 0)),
        pl.BlockSpec((H, H), lambda i: (0, 0)),
        pl.BlockSpec((1, H), lambda i: (0, 0)),
        pl.BlockSpec((1, H), lambda i: (0, 0)),
        pl.BlockSpec((1, H), lambda i: (0, 0)),
        pl.BlockSpec((G, H), lambda i: (0, 0)),
        pl.BlockSpec((1, G), lambda i: (0, 0)),
        pl.BlockSpec((32, G), lambda i: (0, 0)),
        pl.BlockSpec((1, 32), lambda i: (0, 0)),
        pl.BlockSpec((10, 32), lambda i: (0, 0)),
        pl.BlockSpec((1, 10), lambda i: (0, 0)),
    ],
    out_specs=pl.BlockSpec((G, 10), lambda i: (0, 0)),
    out_shape=jax.ShapeDtypeStruct((G, 10), jnp.float32),
    scratch_shapes=[
        pltpu.VMEM((G, H), jnp.float32),
        pltpu.VMEM((G, 1), jnp.float32),
    ],
)


def kernel(node_feat, edge_index, graph_ids, emb, W1, b1, W2, b2,
           We1, be1, g0, bb0, We2, be2, g1, bb1, We3, be3, We4, be4, We5, be5):
    nf2 = jnp.concatenate(
        [node_feat, jnp.zeros((NPAD - N,), jnp.int32)]).reshape(NW, ROWS_PER_W)
    pad_e = N + jnp.arange(EPAD - E, dtype=jnp.int32) % (NPAD - N)
    src3 = jnp.concatenate([edge_index[0], pad_e]).reshape(NW, NSTEPS, ECHUNK)
    dst3 = jnp.concatenate([edge_index[1], pad_e]).reshape(NW, NSTEPS, ECHUNK)
    gid = jnp.concatenate(
        [graph_ids, jnp.full((NPAD - N,), G, jnp.int32)]).reshape(NPAD, 1)

    h0, odp, idp = _prep_call(nf2, emb, src3, dst3)
    m0, osc, isc = _pre1_call(jnp.transpose(odp), jnp.transpose(idp), h0, W1)

    part1 = _agg_call(m0, src3, dst3)
    m1 = _mid_call(part1, isc, osc, b1.reshape(1, H), W2)

    part2 = _agg_call(m1, src3, dst3)
    out = _final_call(part2, isc, gid, b2.reshape(1, H),
                      We1, be1.reshape(1, H), g0.reshape(1, H), bb0.reshape(1, H),
                      We2, be2.reshape(1, H), g1.reshape(1, H), bb1.reshape(1, H),
                      We3, be3.reshape(1, G), We4, be4.reshape(1, 32),
                      We5, be5.reshape(1, 10))
    return out


# speedup vs baseline: 3.4062x; 1.0010x over previous
import functools

import jax
import jax.numpy as jnp
from jax import lax
from jax.experimental import pallas as pl
from jax.experimental.pallas import tpu as pltpu
from jax.experimental.pallas import tpu_sc as plsc

N = 10000
E = 320000
H = 128
G = 64
EPS = 1e-5

NCORE = 2
NSUB = 16
NW = NCORE * NSUB
NPAD = 10240
ROWS_PER_W = NPAD // NW
GCHUNKS = ((0, 128), (128, 128), (256, 64))
EPW = 10240
EPAD = NW * EPW
ECHUNK = 64
NSTEPS = EPW // ECHUNK
NBUF = 4
IBLK = 40
NIBLK = NSTEPS // IBLK
PCHUNK = 128
RPT = NPAD // NSUB
DPT = RPT

_mesh = plsc.VectorSubcoreMesh(core_axis_name="c", subcore_axis_name="s")


def _zero_rows(ref, nrows):
    def body(r, _):
        for k in range(H // 16):
            ref[r, pl.ds(k * 16, 16)] = jnp.zeros((16,), jnp.float32)
        return 0
    lax.fori_loop(0, nrows, body, 0)


def _zero_vec(ref, n):
    def body(r, _):
        ref[pl.ds(r * 16, 16)] = jnp.zeros((16,), jnp.float32)
        return 0
    lax.fori_loop(0, n // 16, body, 0)


def _prep_body(nf2, emb, src3, dst3, h0, odp, idp,
               nf_v, src_v, dst_v, rows_v, ones_v, dbuf_v, od_s, id_s, sem, dsem):
    c = lax.axis_index("c")
    s = lax.axis_index("s")
    wid = c * NSUB + s

    pltpu.sync_copy(nf2.at[wid], nf_v)
    pltpu.sync_copy(src3.at[wid], src_v)
    pltpu.sync_copy(dst3.at[wid], dst_v)

    _zero_vec(dbuf_v, DPT)
    pltpu.sync_copy(dbuf_v, od_s.at[pl.ds(s * DPT, DPT)])
    pltpu.sync_copy(dbuf_v, id_s.at[pl.ds(s * DPT, DPT)])
    for k in range(ECHUNK // 16):
        ones_v[pl.ds(k * 16, 16)] = jnp.full((16,), 1.0, jnp.float32)
    plsc.subcore_barrier()

    def dfire(j, _):
        pltpu.async_copy(ones_v, od_s.at[src_v.at[j]], dsem, add=True)
        pltpu.async_copy(ones_v, id_s.at[dst_v.at[j]], dsem, add=True)
        return 0
    lax.fori_loop(0, NSTEPS, dfire, 0)

    for off, sz in GCHUNKS:
        pltpu.async_copy(emb.at[nf_v.at[pl.ds(off, sz)]],
                         rows_v.at[pl.ds(0, sz)], sem).wait()
        pltpu.sync_copy(rows_v.at[pl.ds(0, sz)],
                        h0.at[pl.ds(wid * ROWS_PER_W + off, sz)])

    def ddrain(j, _):
        pltpu.make_async_copy(ones_v, od_s.at[src_v.at[j]], dsem).wait()
        pltpu.make_async_copy(ones_v, id_s.at[dst_v.at[j]], dsem).wait()
        return 0
    lax.fori_loop(0, NSTEPS, ddrain, 0)

    plsc.subcore_barrier()

    pltpu.sync_copy(od_s.at[pl.ds(s * DPT, DPT)], odp.at[c, pl.ds(s * DPT, DPT)])
    pltpu.sync_copy(id_s.at[pl.ds(s * DPT, DPT)], idp.at[c, pl.ds(s * DPT, DPT)])


_prep_call = pl.kernel(
    _prep_body,
    out_type=(
        jax.ShapeDtypeStruct((NPAD, H), jnp.float32),
        jax.ShapeDtypeStruct((NCORE, NPAD), jnp.float32),
        jax.ShapeDtypeStruct((NCORE, NPAD), jnp.float32),
    ),
    mesh=_mesh,
    scratch_types=[
        pltpu.VMEM((ROWS_PER_W,), jnp.int32),
        pltpu.VMEM((NSTEPS, ECHUNK), jnp.int32),
        pltpu.VMEM((NSTEPS, ECHUNK), jnp.int32),
        pltpu.VMEM((PCHUNK, H), jnp.float32),
        pltpu.VMEM((ECHUNK,), jnp.float32),
        pltpu.VMEM((DPT,), jnp.float32),
        pltpu.VMEM_SHARED((NPAD,), jnp.float32),
        pltpu.VMEM_SHARED((NPAD,), jnp.float32),
        pltpu.SemaphoreType.DMA,
        pltpu.SemaphoreType.DMA,
    ],
)


def _agg_body(hs, src3, dst3, part, src_i, dst_i, rows_v, agg_s, sem):
    c = lax.axis_index("c")
    s = lax.axis_index("s")
    wid = c * NSUB + s

    _zero_rows(rows_v.at[0], ECHUNK)
    for i in range(RPT // ECHUNK):
        pltpu.sync_copy(rows_v.at[0], agg_s.at[pl.ds(s * RPT + i * ECHUNK, ECHUNK)])
    plsc.subcore_barrier()

    for blk in range(NIBLK):
        pltpu.sync_copy(src3.at[wid, pl.ds(blk * IBLK, IBLK)], src_i)
        pltpu.sync_copy(dst3.at[wid, pl.ds(blk * IBLK, IBLK)], dst_i)
        for b in range(NBUF):
            pltpu.async_copy(hs.at[src_i.at[b]], rows_v.at[b], sem)

        def outer(g, _):
            for b in range(NBUF):
                j = g * NBUF + b
                pltpu.make_async_copy(hs.at[src_i.at[j]], rows_v.at[b], sem).wait()
                pltpu.sync_copy(rows_v.at[b], agg_s.at[dst_i.at[j]], add=True)

                @pl.when(j + NBUF < IBLK)
                def _():
                    pltpu.async_copy(hs.at[src_i.at[j + NBUF]], rows_v.at[b], sem)
            return 0
        lax.fori_loop(0, IBLK // NBUF, outer, 0)

    plsc.subcore_barrier()

    pltpu.sync_copy(agg_s.at[pl.ds(s * RPT, RPT)],
                    part.at[c, pl.ds(s * RPT, RPT)])


_agg_call = pl.kernel(
    _agg_body,
    out_type=jax.ShapeDtypeStruct((NCORE, NPAD, H), jnp.float32),
    mesh=_mesh,
    scratch_types=[
        pltpu.VMEM((IBLK, ECHUNK), jnp.int32),
        pltpu.VMEM((IBLK, ECHUNK), jnp.int32),
        pltpu.VMEM((NBUF, ECHUNK, H), jnp.float32),
        pltpu.VMEM_SHARED((NPAD, H), jnp.float32),
        pltpu.SemaphoreType.DMA,
    ],
)


_BLK = 2048
_NBLK = NPAD // _BLK


def _pre1_body(odp_ref, idp_ref, h0_ref, w_ref, m_ref, osc_ref, isc_ref):
    od = odp_ref[...]
    idg = idp_ref[...]
    osc = lax.rsqrt(jnp.maximum(od[:, 0:1] + od[:, 1:2], 1.0))
    isc = lax.rsqrt(jnp.maximum(idg[:, 0:1] + idg[:, 1:2], 1.0))
    m_ref[...] = lax.dot_general(h0_ref[...] * osc, w_ref[...],
                                 (((1,), (0,)), ((), ())),
                                 preferred_element_type=jnp.float32)
    osc_ref[...] = osc
    isc_ref[...] = isc


_pre1_call = pl.pallas_call(
    _pre1_body,
    grid=(_NBLK,),
    in_specs=[
        pl.BlockSpec((_BLK, 2), lambda i: (i, 0)),
        pl.BlockSpec((_BLK, 2), lambda i: (i, 0)),
        pl.BlockSpec((_BLK, H), lambda i: (i, 0)),
        pl.BlockSpec((H, H), lambda i: (0, 0)),
    ],
    out_specs=[
        pl.BlockSpec((_BLK, H), lambda i: (i, 0)),
        pl.BlockSpec((_BLK, 1), lambda i: (i, 0)),
        pl.BlockSpec((_BLK, 1), lambda i: (i, 0)),
    ],
    out_shape=[
        jax.ShapeDtypeStruct((NPAD, H), jnp.float32),
        jax.ShapeDtypeStruct((NPAD, 1), jnp.float32),
        jax.ShapeDtypeStruct((NPAD, 1), jnp.float32),
    ],
)


def _mid_body(part_ref, isc_ref, osc_ref, b_ref, w_ref, out_ref):
    p = part_ref[0] + part_ref[1]
    t = jnp.maximum(p * isc_ref[...] + b_ref[...], 0.0)
    out_ref[...] = lax.dot_general(t * osc_ref[...], w_ref[...],
                                   (((1,), (0,)), ((), ())),
                                   preferred_element_type=jnp.float32)


_mid_call = pl.pallas_call(
    _mid_body,
    grid=(_NBLK,),
    in_specs=[
        pl.BlockSpec((NCORE, _BLK, H), lambda i: (0, i, 0)),
        pl.BlockSpec((_BLK, 1), lambda i: (i, 0)),
        pl.BlockSpec((_BLK, 1), lambda i: (i, 0)),
        pl.BlockSpec((1, H), lambda i: (0, 0)),
        pl.BlockSpec((H, H), lambda i: (0, 0)),
    ],
    out_specs=pl.BlockSpec((_BLK, H), lambda i: (i, 0)),
    out_shape=jax.ShapeDtypeStruct((NPAD, H), jnp.float32),
)


def _leaky(x, slope):
    return jnp.where(x >= 0, x, slope * x)


def _final_body(part_ref, isc_ref, gid_ref, b2_ref,
                we1_ref, be1_ref, g0_ref, bb0_ref,
                we2_ref, be2_ref, g1_ref, bb1_ref,
                we3_ref, be3_ref, we4_ref, be4_ref, we5_ref, be5_ref,
                out_ref, hg_acc, cnt_acc):
    i = pl.program_id(0)

    p = part_ref[0] + part_ref[1]
    h2 = jnp.maximum(p * isc_ref[...] + b2_ref[...], 0.0)

    ids = gid_ref[...]
    seg = lax.broadcasted_iota(jnp.int32, (_BLK, G), 1)
    oh = (ids == seg).astype(jnp.float32)

    @pl.when(i == 0)
    def _():
        hg_acc[...] = jnp.zeros((G, H), jnp.float32)
        cnt_acc[...] = jnp.zeros((G, 1), jnp.float32)

    hg_acc[...] += lax.dot_general(oh, h2, (((0,), (0,)), ((), ())),
                                   preferred_element_type=jnp.float32)
    cnt_acc[...] += lax.dot_general(oh, jnp.ones((_BLK, 1), jnp.float32),
                                    (((0,), (0,)), ((), ())),
                                    preferred_element_type=jnp.float32)

    @pl.when(i == _NBLK - 1)
    def _():
        hg = hg_acc[...] / jnp.maximum(cnt_acc[...], 1.0)
        bninv = 1.0 / jnp.sqrt(1.0 + EPS)

        def lin_t(x, w_ref, b_ref):
            return lax.dot_general(x, w_ref[...], (((1,), (1,)), ((), ())),
                                   preferred_element_type=jnp.float32) + b_ref[...]

        h1 = _leaky(g0_ref[...] * lin_t(hg, we1_ref, be1_ref) * bninv + bb0_ref[...], 0.05)
        hb = _leaky(g1_ref[...] * lin_t(h1, we2_ref, be2_ref) * bninv + bb1_ref[...], 0.05)
        h3 = _leaky(lin_t(hb, we3_ref, be3_ref), 0.1)
        h4 = _leaky(lin_t(h3, we4_ref, be4_ref), 0.1)
        y = lin_t(h4, we5_ref, be5_ref)
        m = jnp.max(y, axis=1, keepdims=True)
        z = y - m
        out_ref[...] = z - jnp.log(jnp.sum(jnp.exp(z), axis=1, keepdims=True))


_final_call = pl.pallas_call(
    _final_body,
    grid=(_NBLK,),
    in_specs=[
        pl.BlockSpec((NCORE, _BLK, H), lambda i: (0, i, 0)),
        pl.BlockSpec((_BLK, 1), lambda i: (i, 0)),
        pl.BlockSpec((_BLK, 1), lambda i: (i, 0)),
        pl.BlockSpec((1, H), lambda i: (0, 0)),
        pl.BlockSpec((H, H), lambda i: (0, 0)),
        pl.BlockSpec((1, H), lambda i: (0, 0)),
        pl.BlockSpec((1, H), lambda i: (0, 0)),
        pl.BlockSpec((1, H), lambda i: (0, 0)),
        pl.BlockSpec((H, H), lambda i: (0, 0)),
        pl.BlockSpec((1, H), lambda i: (0, 0)),
        pl.BlockSpec((1, H), lambda i: (0, 0)),
        pl.BlockSpec((1, H), lambda i: (0, 0)),
        pl.BlockSpec((G, H), lambda i: (0, 0)),
        pl.BlockSpec((1, G), lambda i: (0, 0)),
        pl.BlockSpec((32, G), lambda i: (0, 0)),
        pl.BlockSpec((1, 32), lambda i: (0, 0)),
        pl.BlockSpec((10, 32), lambda i: (0, 0)),
        pl.BlockSpec((1, 10), lambda i: (0, 0)),
    ],
    out_specs=pl.BlockSpec((G, 10), lambda i: (0, 0)),
    out_shape=jax.ShapeDtypeStruct((G, 10), jnp.float32),
    scratch_shapes=[
        pltpu.VMEM((G, H), jnp.float32),
        pltpu.VMEM((G, 1), jnp.float32),
    ],
)


def kernel(node_feat, edge_index, graph_ids, emb, W1, b1, W2, b2,
           We1, be1, g0, bb0, We2, be2, g1, bb1, We3, be3, We4, be4, We5, be5):
    nf2 = jnp.concatenate(
        [node_feat, jnp.zeros((NPAD - N,), jnp.int32)]).reshape(NW, ROWS_PER_W)
    pad_e = N + jnp.arange(EPAD - E, dtype=jnp.int32) % (NPAD - N)
    src3 = jnp.concatenate([edge_index[0], pad_e]).reshape(NW, NSTEPS, ECHUNK)
    dst3 = jnp.concatenate([edge_index[1], pad_e]).reshape(NW, NSTEPS, ECHUNK)
    gid = jnp.concatenate(
        [graph_ids, jnp.full((NPAD - N,), G, jnp.int32)]).reshape(NPAD, 1)

    h0, odp, idp = _prep_call(nf2, emb, src3, dst3)
    m0, osc, isc = _pre1_call(jnp.transpose(odp), jnp.transpose(idp), h0, W1)

    part1 = _agg_call(m0, src3, dst3)
    m1 = _mid_call(part1, isc, osc, b1.reshape(1, H), W2)

    part2 = _agg_call(m1, src3, dst3)
    out = _final_call(part2, isc, gid, b2.reshape(1, H),
                      We1, be1.reshape(1, H), g0.reshape(1, H), bb0.reshape(1, H),
                      We2, be2.reshape(1, H), g1.reshape(1, H), bb1.reshape(1, H),
                      We3, be3.reshape(1, G), We4, be4.reshape(1, 32),
                      We5, be5.reshape(1, 10))
    return out
